# Initial kernel scaffold; baseline (speedup 1.0000x reference)
import jax
import jax.numpy as jnp
from jax.experimental import pallas as pl


def kernel(x, w_attn, b_attn, w_proj, b_proj):
    raise NotImplementedError("write your pallas kernel here")


# speedup vs baseline: 2.0001x; 2.0001x over previous
import functools
import math

import jax
import jax.numpy as jnp
from jax.experimental import pallas as pl
from jax.experimental.pallas import tpu as pltpu

_NEG_BIG = -1e30
_VMEM_LIMIT = 100 * 2**20


def _pick_tile(n, prefs):
    for p in prefs:
        if n % p == 0:
            return p
    return n


def _linear_kernel(x_ref, w_ref, b_ref, o_ref):
    o_ref[...] = (
        jnp.dot(x_ref[...], w_ref[...], preferred_element_type=jnp.float32)
        + b_ref[...]
    ).astype(o_ref.dtype)


def _linear_bf16(x2d, w, b, out_dtype):
    M, K = x2d.shape
    _, N = w.shape
    tm = _pick_tile(M, (1024, 512, 256, 128))
    tn = _pick_tile(N, (768, 512, 256, 128))
    return pl.pallas_call(
        _linear_kernel,
        out_shape=jax.ShapeDtypeStruct((M, N), out_dtype),
        grid=(M // tm, N // tn),
        in_specs=[
            pl.BlockSpec((tm, K), lambda i, j: (i, 0)),
            pl.BlockSpec((K, tn), lambda i, j: (0, j)),
            pl.BlockSpec((1, tn), lambda i, j: (0, j)),
        ],
        out_specs=pl.BlockSpec((tm, tn), lambda i, j: (i, j)),
        compiler_params=pltpu.CompilerParams(
            dimension_semantics=("parallel", "parallel"),
            vmem_limit_bytes=_VMEM_LIMIT,
        ),
        cost_estimate=pl.CostEstimate(
            flops=2 * M * N * K,
            transcendentals=0,
            bytes_accessed=2 * (M * K + K * N + M * N),
        ),
    )(x2d, w, b)


def _attn_proj_kernel(q_ref, k_ref, v_ref, wp_ref, bp_ref, o_ref,
                      acc_ref, m_ref, l_ref, *, H, D, tq, tkv):
    qi = pl.program_id(1)
    ki = pl.program_id(2)

    @pl.when(ki == 0)
    def _init():
        m_ref[...] = jnp.full_like(m_ref, _NEG_BIG)
        l_ref[...] = jnp.zeros_like(l_ref)
        acc_ref[...] = jnp.zeros_like(acc_ref)

    last_needed = (qi * tq + tq - 1) // tkv
    live = ki <= last_needed
    straddles = ki * tkv + tkv - 1 > qi * tq

    def tile(masked):
        if masked:
            row = qi * tq + jax.lax.broadcasted_iota(jnp.int32, (tq, tkv), 0)
            col = ki * tkv + jax.lax.broadcasted_iota(jnp.int32, (tq, tkv), 1)
            hidden = col > row
        for h in range(H):
            sl = slice(h * D, (h + 1) * D)
            s = jax.lax.dot_general(
                q_ref[:, sl], k_ref[:, sl], (((1,), (1,)), ((), ())),
                preferred_element_type=jnp.float32)
            if masked:
                s = jnp.where(hidden, _NEG_BIG, s)
            m_prev = m_ref[:, h:h + 1]
            m_new = jnp.maximum(m_prev, jnp.max(s, axis=-1, keepdims=True))
            alpha = jnp.exp(m_prev - m_new)
            p = jnp.exp(s - m_new)
            l_ref[:, h:h + 1] = (alpha * l_ref[:, h:h + 1]
                                 + jnp.sum(p, axis=-1, keepdims=True))
            acc_ref[:, sl] = alpha * acc_ref[:, sl] + jax.lax.dot_general(
                p.astype(v_ref.dtype), v_ref[:, sl], (((1,), (0,)), ((), ())),
                preferred_element_type=jnp.float32)
            m_ref[:, h:h + 1] = m_new

    @pl.when(jnp.logical_and(live, straddles))
    def _():
        tile(True)

    @pl.when(jnp.logical_and(live, jnp.logical_not(straddles)))
    def _():
        tile(False)

    @pl.when(ki == pl.num_programs(2) - 1)
    def _finalize():
        for h in range(H):
            sl = slice(h * D, (h + 1) * D)
            inv = pl.reciprocal(l_ref[:, h:h + 1], approx=True)
            acc_ref[:, sl] = acc_ref[:, sl] * inv
        y = acc_ref[...].astype(wp_ref.dtype)
        o_ref[...] = (
            jax.lax.dot_general(y, wp_ref[...], (((1,), (0,)), ((), ())),
                                preferred_element_type=jnp.float32)
            + bp_ref[...]
        ).astype(o_ref.dtype)


def _attention_proj(qkv, w_proj, b_proj, *, n_head, out_dtype):
    B, T, C3 = qkv.shape
    C = C3 // 3
    H = n_head
    D = C // H
    tq = _pick_tile(T, (256, 128))
    tkv = _pick_tile(T, (256, 128))
    nq, nkv = T // tq, T // tkv

    kernel_fn = functools.partial(_attn_proj_kernel, H=H, D=D, tq=tq, tkv=tkv)

    def kv_index(b, qi, ki, which):
        return (b, jnp.minimum(ki, (qi * tq + tq - 1) // tkv), which)

    return pl.pallas_call(
        kernel_fn,
        out_shape=jax.ShapeDtypeStruct((B, T, C), out_dtype),
        grid=(B, nq, nkv),
        in_specs=[
            pl.BlockSpec((pl.Squeezed(), tq, C), lambda b, qi, ki: (b, qi, 0)),
            pl.BlockSpec((pl.Squeezed(), tkv, C),
                         lambda b, qi, ki: kv_index(b, qi, ki, 1)),
            pl.BlockSpec((pl.Squeezed(), tkv, C),
                         lambda b, qi, ki: kv_index(b, qi, ki, 2)),
            pl.BlockSpec((C, C), lambda b, qi, ki: (0, 0)),
            pl.BlockSpec((1, C), lambda b, qi, ki: (0, 0)),
        ],
        out_specs=pl.BlockSpec((pl.Squeezed(), tq, C),
                               lambda b, qi, ki: (b, qi, 0)),
        scratch_shapes=[
            pltpu.VMEM((tq, C), jnp.float32),
            pltpu.VMEM((tq, H), jnp.float32),
            pltpu.VMEM((tq, H), jnp.float32),
        ],
        compiler_params=pltpu.CompilerParams(
            dimension_semantics=("parallel", "parallel", "arbitrary"),
            vmem_limit_bytes=_VMEM_LIMIT,
        ),
        cost_estimate=pl.CostEstimate(
            flops=2 * B * T * T * C + 2 * B * T * C * C,
            transcendentals=B * H * T * T // 2,
            bytes_accessed=2 * (B * T * C3 + B * nq * T * 2 * C // 2
                                + B * T * C) + 4 * B * T * C,
        ),
    )(qkv, qkv, qkv, w_proj, b_proj)


def kernel(x, w_attn, b_attn, w_proj, b_proj):
    n_head = 12
    B, T, C = x.shape
    D = C // n_head
    scale = 1.0 / math.sqrt(D)

    col_scale = jnp.where(jnp.arange(3 * C) < C, scale, 1.0)
    w_attn_s = (w_attn * col_scale[None, :]).astype(jnp.bfloat16)
    b_attn_s = (b_attn * col_scale).reshape(1, 3 * C)

    qkv = _linear_bf16(x.reshape(B * T, C).astype(jnp.bfloat16),
                       w_attn_s, b_attn_s, jnp.bfloat16)
    qkv = qkv.reshape(B, T, 3 * C)

    y = _attention_proj(qkv, w_proj.astype(jnp.bfloat16),
                        b_proj.reshape(1, C), n_head=n_head, out_dtype=x.dtype)
    return y


# speedup vs baseline: 2.3401x; 1.1700x over previous
import functools
import math

import jax
import jax.numpy as jnp
from jax.experimental import pallas as pl
from jax.experimental.pallas import tpu as pltpu

_NEG_BIG = -1e30
_VMEM_LIMIT = 100 * 2**20


def _pick_tile(n, prefs):
    for p in prefs:
        if n % p == 0:
            return p
    return n


def _linear_kernel(x_ref, w_ref, b_ref, o_ref):
    o_ref[...] = (
        jnp.dot(x_ref[...], w_ref[...], preferred_element_type=jnp.float32)
        + b_ref[...]
    ).astype(o_ref.dtype)


def _linear_bf16(x2d, w, b, out_dtype):
    M, K = x2d.shape
    _, N = w.shape
    tm = _pick_tile(M, (1024, 512, 256, 128))
    tn = _pick_tile(N, (768, 512, 256, 128))
    return pl.pallas_call(
        _linear_kernel,
        out_shape=jax.ShapeDtypeStruct((M, N), out_dtype),
        grid=(M // tm, N // tn),
        in_specs=[
            pl.BlockSpec((tm, K), lambda i, j: (i, 0)),
            pl.BlockSpec((K, tn), lambda i, j: (0, j)),
            pl.BlockSpec((1, tn), lambda i, j: (0, j)),
        ],
        out_specs=pl.BlockSpec((tm, tn), lambda i, j: (i, j)),
        compiler_params=pltpu.CompilerParams(
            dimension_semantics=("parallel", "parallel"),
            vmem_limit_bytes=_VMEM_LIMIT,
        ),
        cost_estimate=pl.CostEstimate(
            flops=2 * M * N * K,
            transcendentals=0,
            bytes_accessed=2 * (M * K + K * N + M * N),
        ),
    )(x2d, w, b)


def _attn_proj_kernel(q_ref, k_ref, v_ref, wp_ref, bp_ref, o_ref,
                      acc_ref, m_ref, l_ref, *, H, D, tq, tkv):
    qi = pl.program_id(1)
    ki = pl.program_id(2)

    @pl.when(ki == 0)
    def _init():
        m_ref[...] = jnp.full_like(m_ref, _NEG_BIG)
        l_ref[...] = jnp.zeros_like(l_ref)
        acc_ref[...] = jnp.zeros_like(acc_ref)

    last_needed = (qi * tq + tq - 1) // tkv
    live = ki <= last_needed
    straddles = ki * tkv + tkv - 1 > qi * tq

    def tile(masked):
        if masked:
            row = qi * tq + jax.lax.broadcasted_iota(jnp.int32, (tq, tkv), 0)
            col = ki * tkv + jax.lax.broadcasted_iota(jnp.int32, (tq, tkv), 1)
            hidden = col > row
        ones_col = jnp.ones((tkv, 1), dtype=v_ref.dtype)
        for h in range(H):
            sl = slice(h * D, (h + 1) * D)
            s = jax.lax.dot_general(
                q_ref[:, sl], k_ref[:, sl], (((1,), (1,)), ((), ())),
                preferred_element_type=jnp.float32)
            if masked:
                s = jnp.where(hidden, _NEG_BIG, s)
            m_prev = m_ref[:, h:h + 1]
            m_new = jnp.maximum(m_prev, jnp.max(s, axis=-1, keepdims=True))
            alpha = jnp.exp2(m_prev - m_new)
            p = jnp.exp2(s - m_new)
            v_aug = jnp.concatenate([v_ref[:, sl], ones_col], axis=1)
            pv = jax.lax.dot_general(
                p.astype(v_ref.dtype), v_aug, (((1,), (0,)), ((), ())),
                preferred_element_type=jnp.float32)
            acc_ref[:, sl] = alpha * acc_ref[:, sl] + pv[:, :D]
            l_ref[:, h:h + 1] = alpha * l_ref[:, h:h + 1] + pv[:, D:D + 1]
            m_ref[:, h:h + 1] = m_new

    @pl.when(jnp.logical_and(live, straddles))
    def _():
        tile(True)

    @pl.when(jnp.logical_and(live, jnp.logical_not(straddles)))
    def _():
        tile(False)

    @pl.when(ki == pl.num_programs(2) - 1)
    def _finalize():
        for h in range(H):
            sl = slice(h * D, (h + 1) * D)
            inv = pl.reciprocal(l_ref[:, h:h + 1], approx=True)
            acc_ref[:, sl] = acc_ref[:, sl] * inv
        y = acc_ref[...].astype(wp_ref.dtype)
        o_ref[...] = (
            jax.lax.dot_general(y, wp_ref[...], (((1,), (0,)), ((), ())),
                                preferred_element_type=jnp.float32)
            + bp_ref[...]
        ).astype(o_ref.dtype)


def _attention_proj(qkv, w_proj, b_proj, *, n_head, out_dtype):
    B, T, C3 = qkv.shape
    C = C3 // 3
    H = n_head
    D = C // H
    tq = _pick_tile(T, (256, 128))
    tkv = _pick_tile(T, (256, 128))
    nq, nkv = T // tq, T // tkv

    kernel_fn = functools.partial(_attn_proj_kernel, H=H, D=D, tq=tq, tkv=tkv)

    def kv_index(b, qi, ki, which):
        return (b, jnp.minimum(ki, (qi * tq + tq - 1) // tkv), which)

    return pl.pallas_call(
        kernel_fn,
        out_shape=jax.ShapeDtypeStruct((B, T, C), out_dtype),
        grid=(B, nq, nkv),
        in_specs=[
            pl.BlockSpec((pl.Squeezed(), tq, C), lambda b, qi, ki: (b, qi, 0)),
            pl.BlockSpec((pl.Squeezed(), tkv, C),
                         lambda b, qi, ki: kv_index(b, qi, ki, 1)),
            pl.BlockSpec((pl.Squeezed(), tkv, C),
                         lambda b, qi, ki: kv_index(b, qi, ki, 2)),
            pl.BlockSpec((C, C), lambda b, qi, ki: (0, 0)),
            pl.BlockSpec((1, C), lambda b, qi, ki: (0, 0)),
        ],
        out_specs=pl.BlockSpec((pl.Squeezed(), tq, C),
                               lambda b, qi, ki: (b, qi, 0)),
        scratch_shapes=[
            pltpu.VMEM((tq, C), jnp.float32),
            pltpu.VMEM((tq, H), jnp.float32),
            pltpu.VMEM((tq, H), jnp.float32),
        ],
        compiler_params=pltpu.CompilerParams(
            dimension_semantics=("parallel", "parallel", "arbitrary"),
            vmem_limit_bytes=_VMEM_LIMIT,
        ),
        cost_estimate=pl.CostEstimate(
            flops=2 * B * T * T * C + 2 * B * T * C * C,
            transcendentals=B * H * T * T // 2,
            bytes_accessed=2 * (B * T * C3 + B * nq * T * 2 * C // 2
                                + B * T * C) + 4 * B * T * C,
        ),
    )(qkv, qkv, qkv, w_proj, b_proj)


def kernel(x, w_attn, b_attn, w_proj, b_proj):
    n_head = 12
    B, T, C = x.shape
    D = C // n_head
    scale = 1.0 / math.sqrt(D)

    col_scale = jnp.where(jnp.arange(3 * C) < C, scale * math.log2(math.e), 1.0)
    w_attn_s = (w_attn * col_scale[None, :]).astype(jnp.bfloat16)
    b_attn_s = (b_attn * col_scale).reshape(1, 3 * C)

    qkv = _linear_bf16(x.reshape(B * T, C).astype(jnp.bfloat16),
                       w_attn_s, b_attn_s, jnp.bfloat16)
    qkv = qkv.reshape(B, T, 3 * C)

    y = _attention_proj(qkv, w_proj.astype(jnp.bfloat16),
                        b_proj.reshape(1, C), n_head=n_head, out_dtype=x.dtype)
    return y


# speedup vs baseline: 4.8903x; 2.0898x over previous
import functools
import math

import jax
import jax.numpy as jnp
from jax.experimental import pallas as pl
from jax.experimental.pallas import tpu as pltpu

_VMEM_LIMIT = 100 * 2**20


def _pick_tile(n, prefs):
    for p in prefs:
        if n % p == 0:
            return p
    return n


def _qkv_proj_kernel(x_ref, w_ref, b_ref, o_ref, qn_ref, kn_ref, *, H, D):
    j = pl.program_id(1)
    y = jnp.dot(x_ref[...], w_ref[...],
                preferred_element_type=jnp.float32) + b_ref[...]
    o_ref[...] = y.astype(o_ref.dtype)

    def _norms(n_ref):
        cols = []
        for h in range(H):
            yh = y[:, h * D:(h + 1) * D]
            cols.append(jnp.sum(yh * yh, axis=1, keepdims=True))
        n_ref[...] = jnp.concatenate(cols, axis=1)

    @pl.when(j == 0)
    def _():
        _norms(qn_ref)

    @pl.when(j == 1)
    def _():
        _norms(kn_ref)


def _qkv_projection(x2d, w, b, *, H, D):
    M, K = x2d.shape
    _, N = w.shape
    C = N // 3
    tm = _pick_tile(M, (1024, 512, 256, 128))
    return pl.pallas_call(
        functools.partial(_qkv_proj_kernel, H=H, D=D),
        out_shape=(
            jax.ShapeDtypeStruct((M, N), jnp.bfloat16),
            jax.ShapeDtypeStruct((M, H), jnp.float32),
            jax.ShapeDtypeStruct((M, H), jnp.float32),
        ),
        grid=(M // tm, 3),
        in_specs=[
            pl.BlockSpec((tm, K), lambda i, j: (i, 0)),
            pl.BlockSpec((K, C), lambda i, j: (0, j)),
            pl.BlockSpec((1, C), lambda i, j: (0, j)),
        ],
        out_specs=(
            pl.BlockSpec((tm, C), lambda i, j: (i, j)),
            pl.BlockSpec((tm, H), lambda i, j: (i, 0)),
            pl.BlockSpec((tm, H), lambda i, j: (i, 0)),
        ),
        compiler_params=pltpu.CompilerParams(
            dimension_semantics=("parallel", "parallel"),
            vmem_limit_bytes=_VMEM_LIMIT,
        ),
        cost_estimate=pl.CostEstimate(
            flops=2 * M * N * K,
            transcendentals=0,
            bytes_accessed=2 * (M * K + K * N + M * N),
        ),
    )(x2d, w, b)


def _attn_proj_kernel(q_ref, k_ref, v_ref, qn_ref, g_ref, wp_ref, bp_ref,
                      o_ref, qa_ref, acc_ref, y_ref, *, H, D, tq, tkv):
    qi = pl.program_id(1)
    ki = pl.program_id(2)

    @pl.when(ki == 0)
    def _init():
        acc_ref[...] = jnp.zeros_like(acc_ref)
        for h in range(H):
            qa_ref[:, h * 128:h * 128 + D] = q_ref[:, h * D:(h + 1) * D]
            m = qn_ref[:, h:h + 1] * g_ref[:, h:h + 1]
            qa_ref[:, h * 128 + D:h * 128 + D + 1] = m.astype(qa_ref.dtype)

    last_needed = (qi * tq + tq - 1) // tkv
    live = ki <= last_needed
    straddles = ki * tkv + tkv - 1 > qi * tq

    def tile(masked):
        if masked:
            row = qi * tq + jax.lax.broadcasted_iota(jnp.int32, (tq, tkv), 0)
            col = ki * tkv + jax.lax.broadcasted_iota(jnp.int32, (tq, tkv), 1)
            hidden = col > row
        neg_col = jnp.full((tkv, 1), -1.0, dtype=k_ref.dtype)
        one_col = jnp.ones((tkv, 1), dtype=v_ref.dtype)
        for h in range(H):
            sl = slice(h * D, (h + 1) * D)
            sa = slice(h * 128, h * 128 + D + 1)
            k_aug = jnp.concatenate([k_ref[:, sl], neg_col], axis=1)
            s = jax.lax.dot_general(
                qa_ref[:, sa], k_aug, (((1,), (1,)), ((), ())),
                preferred_element_type=jnp.float32)
            p = jnp.exp2(s)
            if masked:
                p = jnp.where(hidden, 0.0, p)
            v_aug = jnp.concatenate([v_ref[:, sl], one_col], axis=1)
            pv = jax.lax.dot_general(
                p.astype(v_ref.dtype), v_aug, (((1,), (0,)), ((), ())),
                preferred_element_type=jnp.float32)
            acc_ref[:, sa] = acc_ref[:, sa] + pv

    @pl.when(jnp.logical_and(live, straddles))
    def _():
        tile(True)

    @pl.when(jnp.logical_and(live, jnp.logical_not(straddles)))
    def _():
        tile(False)

    @pl.when(ki == pl.num_programs(2) - 1)
    def _finalize():
        for h in range(H):
            inv = pl.reciprocal(acc_ref[:, h * 128 + D:h * 128 + D + 1],
                                approx=True)
            y_ref[:, h * D:(h + 1) * D] = (
                acc_ref[:, h * 128:h * 128 + D] * inv).astype(y_ref.dtype)
        o_ref[...] = (
            jax.lax.dot_general(y_ref[...], wp_ref[...],
                                (((1,), (0,)), ((), ())),
                                preferred_element_type=jnp.float32)
            + bp_ref[...]
        ).astype(o_ref.dtype)


def _attention_proj(qkv, qnorm, gbound, w_proj, b_proj, *, n_head, out_dtype):
    B, T, C3 = qkv.shape
    C = C3 // 3
    H = n_head
    D = C // H
    tq = _pick_tile(T, (256, 128))
    tkv = _pick_tile(T, (256, 128))
    nq, nkv = T // tq, T // tkv

    kernel_fn = functools.partial(_attn_proj_kernel, H=H, D=D, tq=tq, tkv=tkv)

    def kv_index(b, qi, ki, which):
        return (b, jnp.minimum(ki, (qi * tq + tq - 1) // tkv), which)

    return pl.pallas_call(
        kernel_fn,
        out_shape=jax.ShapeDtypeStruct((B, T, C), out_dtype),
        grid=(B, nq, nkv),
        in_specs=[
            pl.BlockSpec((pl.Squeezed(), tq, C), lambda b, qi, ki: (b, qi, 0)),
            pl.BlockSpec((pl.Squeezed(), tkv, C),
                         lambda b, qi, ki: kv_index(b, qi, ki, 1)),
            pl.BlockSpec((pl.Squeezed(), tkv, C),
                         lambda b, qi, ki: kv_index(b, qi, ki, 2)),
            pl.BlockSpec((pl.Squeezed(), tq, H), lambda b, qi, ki: (b, qi, 0)),
            pl.BlockSpec((pl.Squeezed(), 1, H), lambda b, qi, ki: (b, 0, 0)),
            pl.BlockSpec((C, C), lambda b, qi, ki: (0, 0)),
            pl.BlockSpec((1, C), lambda b, qi, ki: (0, 0)),
        ],
        out_specs=pl.BlockSpec((pl.Squeezed(), tq, C),
                               lambda b, qi, ki: (b, qi, 0)),
        scratch_shapes=[
            pltpu.VMEM((tq, H * 128), jnp.bfloat16),
            pltpu.VMEM((tq, H * 128), jnp.float32),
            pltpu.VMEM((tq, C), jnp.bfloat16),
        ],
        compiler_params=pltpu.CompilerParams(
            dimension_semantics=("parallel", "parallel", "arbitrary"),
            vmem_limit_bytes=_VMEM_LIMIT,
        ),
        cost_estimate=pl.CostEstimate(
            flops=2 * B * T * T * C + 2 * B * T * C * C,
            transcendentals=B * H * T * T // 2,
            bytes_accessed=2 * (B * T * C3 + B * nq * T * 2 * C // 2
                                + B * T * C) + 4 * B * T * C,
        ),
    )(qkv, qkv, qkv, qnorm, gbound, w_proj, b_proj)


def kernel(x, w_attn, b_attn, w_proj, b_proj):
    n_head = 12
    B, T, C = x.shape
    D = C // n_head
    scale = 1.0 / math.sqrt(D)

    col_scale = jnp.where(jnp.arange(3 * C) < C, scale * math.log2(math.e), 1.0)
    w_attn_s = (w_attn * col_scale[None, :]).astype(jnp.bfloat16)
    b_attn_s = (b_attn * col_scale).reshape(1, 3 * C)

    qkv, qn2, kn2 = _qkv_projection(
        x.reshape(B * T, C).astype(jnp.bfloat16), w_attn_s, b_attn_s,
        H=n_head, D=D)
    qkv = qkv.reshape(B, T, 3 * C)

    qnorm = jnp.sqrt(qn2).reshape(B, T, n_head)
    gbound = (jnp.sqrt(jnp.max(kn2.reshape(B, T, n_head), axis=1)) * 1.01
              ).reshape(B, 1, n_head)

    y = _attention_proj(qkv, qnorm, gbound, w_proj.astype(jnp.bfloat16),
                        b_proj.reshape(1, C), n_head=n_head, out_dtype=x.dtype)
    return y
